# Initial kernel scaffold; baseline (speedup 1.0000x reference)
import jax
import jax.numpy as jnp
from jax.experimental import pallas as pl


def kernel(xyz, xyz_edge_index, xyz_edge_attr, batch, W_node, b_node, W_edge, b_edge, eps, W1, b1, mlp_bn_gamma, mlp_bn_beta, W2, b2, bn_gamma, bn_beta, W_pred, b_pred):
    raise NotImplementedError("write your pallas kernel here")


# speedup vs baseline: 1.4623x; 1.4623x over previous
import functools

import jax
import jax.numpy as jnp
from jax import lax
from jax.experimental import pallas as pl
from jax.experimental.pallas import tpu as pltpu
from jax.experimental.pallas import tpu_sc as plsc

N = 50000
E = 800000
EMB = 64
HALF = 32
EDGE_DIM = 16
NLAYERS = 5
G = 128

NC = 2
NS = 16
CH = 128
NCHUNK = E // CH
CPS = -(-NCHUNK // NS)
RB = 80
NRB = N // RB
RBPS = -(-NRB // NS)

BLK_N = 10000
BLK_E = 10000


def _sc_body(hcat, emb, src, dst, order, out, agg_sh, srcv, gidx, didx, ordv,
             eidx, grows, embv, zbuf, sem, sem2):
    c = lax.axis_index("c")
    s = lax.axis_index("s")
    coff = c * N

    @pl.loop(0, RB)
    def _zb(r):
        zbuf[r, pl.ds(0, 16)] = jnp.zeros((16,), jnp.float32)
        zbuf[r, pl.ds(16, 16)] = jnp.zeros((16,), jnp.float32)

    @pl.loop(0, RBPS)
    def _z(k):
        j = s + k * NS

        @pl.when(j < NRB)
        def _():
            pltpu.sync_copy(zbuf, agg_sh.at[pl.ds(j * RB, RB)])

    plsc.subcore_barrier()

    @pl.loop(0, CPS)
    def _chunk(k):
        j = s + k * NS

        @pl.when(j < NCHUNK)
        def _():
            base = j * CH
            pltpu.sync_copy(src.at[pl.ds(base, CH)], srcv)
            pltpu.sync_copy(dst.at[pl.ds(base, CH)], didx.at[0])
            pltpu.sync_copy(order.at[pl.ds(base, CH)], ordv)

            eoff = c * E

            @pl.loop(0, CH // 16)
            def _off(i):
                gidx[pl.ds(i * 16, 16)] = srcv[pl.ds(i * 16, 16)] + coff
                eidx[pl.ds(i * 16, 16)] = ordv[pl.ds(i * 16, 16)] + eoff

            gth = pltpu.async_copy(hcat.at[gidx], grows, sem)
            gte = pltpu.async_copy(emb.at[eidx], embv, sem2)
            gth.wait()
            gte.wait()

            @pl.loop(0, CH, unroll=8)
            def _msg(r):
                a0 = grows[r, pl.ds(0, 16)] + embv[r, pl.ds(0, 16)]
                embv[r, pl.ds(0, 16)] = jnp.maximum(a0, 0.0)
                a1 = grows[r, pl.ds(16, 16)] + embv[r, pl.ds(16, 16)]
                embv[r, pl.ds(16, 16)] = jnp.maximum(a1, 0.0)

            pltpu.sync_copy(embv, agg_sh.at[didx.at[0]], add=True)

    plsc.subcore_barrier()

    @pl.loop(0, RBPS)
    def _wb(k):
        j = s + k * NS

        @pl.when(j < NRB)
        def _():
            pltpu.sync_copy(agg_sh.at[pl.ds(j * RB, RB)],
                            out.at[pl.ds(coff + j * RB, RB)])


@functools.lru_cache(maxsize=None)
def _make_sc_msg_agg():
  return functools.partial(
    pl.kernel,
    out_type=jax.ShapeDtypeStruct((2 * N, HALF), jnp.float32),
    mesh=plsc.VectorSubcoreMesh(core_axis_name="c", subcore_axis_name="s",
                                num_cores=NC, num_subcores=NS),
    scratch_types=[
        pltpu.VMEM_SHARED((N, HALF), jnp.float32),
        pltpu.VMEM((CH,), jnp.int32),
        pltpu.VMEM((CH,), jnp.int32),
        pltpu.VMEM((1, CH), jnp.int32),
        pltpu.VMEM((CH,), jnp.int32),
        pltpu.VMEM((CH,), jnp.int32),
        pltpu.VMEM((CH, HALF), jnp.float32),
        pltpu.VMEM((CH, HALF), jnp.float32),
        pltpu.VMEM((RB, HALF), jnp.float32),
        pltpu.SemaphoreType.DMA,
        pltpu.SemaphoreType.DMA,
    ],
    compiler_params=pltpu.CompilerParams(use_tc_tiling_on_sc=False),
  )(_sc_body)


def _enc_body(xyz_ref, w_ref, b_ref, out_ref):
    h = jnp.dot(xyz_ref[...], w_ref[...],
                preferred_element_type=jnp.float32) + b_ref[...]
    out_ref[0] = h[:, :HALF]
    out_ref[1] = h[:, HALF:]


def _encoder(xyz, w, b):
    return pl.pallas_call(
        _enc_body,
        grid=(N // BLK_N,),
        in_specs=[
            pl.BlockSpec((BLK_N, 3), lambda i: (i, 0)),
            pl.BlockSpec((3, EMB), lambda i: (0, 0)),
            pl.BlockSpec((1, EMB), lambda i: (0, 0)),
        ],
        out_specs=pl.BlockSpec((2, BLK_N, HALF), lambda i: (0, i, 0)),
        out_shape=jax.ShapeDtypeStruct((2, N, HALF), jnp.float32),
    )(xyz, w, b)


def _edge_body(attr_ref, w_ref, b_ref, out_ref):
    emb = jnp.dot(attr_ref[...], w_ref[...],
                  preferred_element_type=jnp.float32) + b_ref[...]
    out_ref[0] = emb[:, :HALF]
    out_ref[1] = emb[:, HALF:]


def _edge_emb(attr, w, b):
    return pl.pallas_call(
        _edge_body,
        grid=(E // BLK_E,),
        in_specs=[
            pl.BlockSpec((BLK_E, EDGE_DIM), lambda i: (i, 0)),
            pl.BlockSpec((EDGE_DIM, EMB), lambda i: (0, 0)),
            pl.BlockSpec((1, EMB), lambda i: (0, 0)),
        ],
        out_specs=pl.BlockSpec((2, BLK_E, HALF), lambda i: (0, i, 0)),
        out_shape=jax.ShapeDtypeStruct((2, E, HALF), jnp.float32),
    )(attr, w, b)


def _pre_t(hlo, hhi, a_ref, eps_ref, w1_ref, b1_ref):
    h = jnp.concatenate([hlo[0], hhi[0]], axis=1)
    pre = (1.0 + eps_ref[0, 0]) * h + a_ref[...]
    t = jnp.dot(pre, w1_ref[...], preferred_element_type=jnp.float32)
    return t + b1_ref[...]


_SPECS_HA = [
    pl.BlockSpec((1, BLK_N, HALF), lambda i: (0, i, 0)),
    pl.BlockSpec((1, BLK_N, HALF), lambda i: (1, i, 0)),
    pl.BlockSpec((BLK_N, EMB), lambda i: (i, 0)),
    pl.BlockSpec((1, 1), lambda i: (0, 0)),
    pl.BlockSpec((EMB, 2 * EMB), lambda i: (0, 0)),
    pl.BlockSpec((1, 2 * EMB), lambda i: (0, 0)),
]
_SPEC_R128 = pl.BlockSpec((1, 2 * EMB), lambda i: (0, 0))
_SPEC_R64 = pl.BlockSpec((1, EMB), lambda i: (0, 0))


def _passA1_body(hlo, hhi, a_ref, eps_ref, w1_ref, b1_ref, sum_ref):
    t = _pre_t(hlo, hhi, a_ref, eps_ref, w1_ref, b1_ref)
    st = jnp.sum(t, axis=0, keepdims=True)

    @pl.when(pl.program_id(0) == 0)
    def _():
        sum_ref[...] = st

    @pl.when(pl.program_id(0) != 0)
    def _():
        sum_ref[...] += st


def _passA1(h_split, agg, eps_l, w1, b1):
    return pl.pallas_call(
        _passA1_body,
        grid=(N // BLK_N,),
        in_specs=list(_SPECS_HA),
        out_specs=_SPEC_R128,
        out_shape=jax.ShapeDtypeStruct((1, 2 * EMB), jnp.float32),
    )(h_split, h_split, agg, eps_l, w1, b1)


def _passA2_body(hlo, hhi, a_ref, eps_ref, w1_ref, b1_ref, ts_ref, var_ref):
    t = _pre_t(hlo, hhi, a_ref, eps_ref, w1_ref, b1_ref)
    d = t - ts_ref[...] / N
    st = jnp.sum(d * d, axis=0, keepdims=True)

    @pl.when(pl.program_id(0) == 0)
    def _():
        var_ref[...] = st

    @pl.when(pl.program_id(0) != 0)
    def _():
        var_ref[...] += st


def _passA2(h_split, agg, eps_l, w1, b1, tsum):
    return pl.pallas_call(
        _passA2_body,
        grid=(N // BLK_N,),
        in_specs=list(_SPECS_HA) + [_SPEC_R128],
        out_specs=_SPEC_R128,
        out_shape=jax.ShapeDtypeStruct((1, 2 * EMB), jnp.float32),
    )(h_split, h_split, agg, eps_l, w1, b1, tsum)


def _passB_body(hlo, hhi, a_ref, eps_ref, w1_ref, b1_ref, tsum_ref,
                tvs_ref, g1_ref, bt1_ref, w2_ref, b2_ref, h2_ref, sum_ref):
    t = _pre_t(hlo, hhi, a_ref, eps_ref, w1_ref, b1_ref)
    m = tsum_ref[...] / N
    sq = jnp.sqrt(tvs_ref[...] / N + 1e-5)
    u = jnp.maximum((t - m) / sq * g1_ref[...] + bt1_ref[...], 0.0)
    h2 = jnp.dot(u, w2_ref[...], preferred_element_type=jnp.float32)
    h2_ref[...] = h2 + b2_ref[...]
    st = jnp.sum(h2 + b2_ref[...], axis=0, keepdims=True)

    @pl.when(pl.program_id(0) == 0)
    def _():
        sum_ref[...] = st

    @pl.when(pl.program_id(0) != 0)
    def _():
        sum_ref[...] += st


def _passB(h_split, agg, eps_l, w1, b1, tsum, tvs, g1, bt1, w2, b2):
    return pl.pallas_call(
        _passB_body,
        grid=(N // BLK_N,),
        in_specs=list(_SPECS_HA) + [
            _SPEC_R128, _SPEC_R128, _SPEC_R128, _SPEC_R128,
            pl.BlockSpec((2 * EMB, EMB), lambda i: (0, 0)),
            _SPEC_R64,
        ],
        out_specs=[
            pl.BlockSpec((BLK_N, EMB), lambda i: (i, 0)),
            _SPEC_R64,
        ],
        out_shape=[
            jax.ShapeDtypeStruct((N, EMB), jnp.float32),
            jax.ShapeDtypeStruct((1, EMB), jnp.float32),
        ],
    )(h_split, h_split, agg, eps_l, w1, b1, tsum, tvs, g1, bt1, w2, b2)


def _h2var_body(h2_ref, sum_ref, var_ref):
    d = h2_ref[...] - sum_ref[...] / N
    st = jnp.sum(d * d, axis=0, keepdims=True)

    @pl.when(pl.program_id(0) == 0)
    def _():
        var_ref[...] = st

    @pl.when(pl.program_id(0) != 0)
    def _():
        var_ref[...] += st


def _h2var(h2, h2sum):
    return pl.pallas_call(
        _h2var_body,
        grid=(N // BLK_N,),
        in_specs=[pl.BlockSpec((BLK_N, EMB), lambda i: (i, 0)), _SPEC_R64],
        out_specs=_SPEC_R64,
        out_shape=jax.ShapeDtypeStruct((1, EMB), jnp.float32),
    )(h2, h2sum)


def _passC_body(h2_ref, sum_ref, var_ref, g_ref, bt_ref, out_ref, *, relu):
    m = sum_ref[...] / N
    sq = jnp.sqrt(var_ref[...] / N + 1e-5)
    hn = (h2_ref[...] - m) / sq * g_ref[...] + bt_ref[...]
    if relu:
        hn = jnp.maximum(hn, 0.0)
    out_ref[0] = hn[:, :HALF]
    out_ref[1] = hn[:, HALF:]


def _passC(h2, h2sum, h2vs, g, bt, relu):
    return pl.pallas_call(
        functools.partial(_passC_body, relu=relu),
        grid=(N // BLK_N,),
        in_specs=[
            pl.BlockSpec((BLK_N, EMB), lambda i: (i, 0)),
            _SPEC_R64, _SPEC_R64, _SPEC_R64, _SPEC_R64,
        ],
        out_specs=pl.BlockSpec((2, BLK_N, HALF), lambda i: (0, i, 0)),
        out_shape=jax.ShapeDtypeStruct((2, N, HALF), jnp.float32),
    )(h2, h2sum, h2vs, g, bt)


def _pool_body(batch_ref, hlo, hhi, wp_ref, bp_ref, out_ref, acc_ref):
    i = pl.program_id(0)
    gids = lax.broadcasted_iota(jnp.int32, (G, BLK_N), 0)
    mask = (batch_ref[0] == gids).astype(jnp.float32)
    h = jnp.concatenate([hlo[0], hhi[0]], axis=1)
    part = jnp.dot(mask, h, preferred_element_type=jnp.float32, precision=lax.Precision.HIGHEST)

    @pl.when(i == 0)
    def _():
        acc_ref[...] = part

    @pl.when(i != 0)
    def _():
        acc_ref[...] += part

    o = jax.nn.sigmoid(
        jnp.dot(acc_ref[...], wp_ref[...], preferred_element_type=jnp.float32)
        + bp_ref[...])
    out_ref[...] = jnp.clip(o, 0.0, 20.0)


def _pool(batch3, h_split, wp, bp):
    return pl.pallas_call(
        _pool_body,
        grid=(N // BLK_N,),
        in_specs=[
            pl.BlockSpec((1, 1, BLK_N), lambda i: (i, 0, 0)),
            pl.BlockSpec((1, BLK_N, HALF), lambda i: (0, i, 0)),
            pl.BlockSpec((1, BLK_N, HALF), lambda i: (1, i, 0)),
            pl.BlockSpec((EMB, 1), lambda i: (0, 0)),
            pl.BlockSpec((1, 1), lambda i: (0, 0)),
        ],
        out_specs=pl.BlockSpec((G, 1), lambda i: (0, 0)),
        out_shape=jax.ShapeDtypeStruct((G, 1), jnp.float32),
        scratch_shapes=[pltpu.VMEM((G, EMB), jnp.float32)],
    )(batch3, h_split, h_split, wp, bp)


def kernel(xyz, xyz_edge_index, xyz_edge_attr, batch, W_node, b_node, W_edge,
           b_edge, eps, W1, b1, mlp_bn_gamma, mlp_bn_beta, W2, b2, bn_gamma,
           bn_beta, W_pred, b_pred):
    src = xyz_edge_index[0].astype(jnp.int32)
    dst = xyz_edge_index[1].astype(jnp.int32)
    order = jnp.argsort(dst, stable=True).astype(jnp.int32)
    src = src[order]
    dst = dst[order]
    batch3 = batch.astype(jnp.int32).reshape(N // BLK_N, 1, BLK_N)

    h_split = _encoder(xyz, W_node, b_node.reshape(1, EMB))
    for l in range(NLAYERS):
        emb = _edge_emb(xyz_edge_attr, W_edge[l], b_edge[l].reshape(1, EMB))
        agg = _make_sc_msg_agg()(h_split.reshape(2 * N, HALF),
                                 emb.reshape(2 * E, HALF), src, dst, order)
        agg = jnp.concatenate([agg[:N], agg[N:]], axis=1)
        eps_l = eps[l].reshape(1, 1)
        b1_l = b1[l].reshape(1, 2 * EMB)
        tsum = _passA1(h_split, agg, eps_l, W1[l], b1_l)
        tvs = _passA2(h_split, agg, eps_l, W1[l], b1_l, tsum)
        h2, h2sum = _passB(h_split, agg, eps_l, W1[l], b1_l, tsum, tvs,
                           mlp_bn_gamma[l].reshape(1, 2 * EMB),
                           mlp_bn_beta[l].reshape(1, 2 * EMB), W2[l],
                           b2[l].reshape(1, EMB))
        h2vs = _h2var(h2, h2sum)
        h_split = _passC(h2, h2sum, h2vs, bn_gamma[l].reshape(1, EMB),
                         bn_beta[l].reshape(1, EMB), relu=(l < NLAYERS - 1))
    return _pool(batch3, h_split, W_pred, b_pred.reshape(1, 1))
